# baseline (speedup 1.0000x reference)
import functools

import jax
import jax.numpy as jnp
from jax import lax
from jax.experimental import pallas as pl
from jax.experimental.pallas import tpu as pltpu
from jax.experimental.pallas import tpu_sc as plsc

NUM_NODES = 10000
NODE_NF = 128
GLOBAL_NF = 64
HIDDEN_NF = 256
NUM_GRAPHS = 64

NC = 2
NS = 16
NW = NC * NS
LANES = 16
COLB = NODE_NF // LANES

N_LO = 24
SG_LO = 10
SG_HI = 9
ROWS_LO = SG_LO * 32
TAIL_BASE = 24 * 320 + 8 * 288
TAIL = NUM_NODES - TAIL_BASE


def _sc_segment_partials(x, batch_i32):
    mesh = plsc.VectorSubcoreMesh(core_axis_name="c", subcore_axis_name="s")

    @functools.partial(
        pl.kernel,
        mesh=mesh,
        out_type=[
            jax.ShapeDtypeStruct((NW, NUM_GRAPHS, NODE_NF), jnp.float32),
            jax.ShapeDtypeStruct((NW, NUM_GRAPHS, LANES), jnp.float32),
        ],
        scratch_types=[
            pltpu.VMEM((2, 32, NODE_NF), jnp.float32),
            pltpu.VMEM((TAIL, NODE_NF), jnp.float32),
            pltpu.VMEM((ROWS_LO + TAIL,), jnp.int32),
            pltpu.VMEM((NUM_GRAPHS, NODE_NF), jnp.float32),
            pltpu.VMEM((NUM_GRAPHS, LANES), jnp.float32),
            pltpu.VMEM((COLB + 1, LANES), jnp.float32),
            pltpu.SMEM((8,), jnp.int32),
            pltpu.SemaphoreType.DMA,
            pltpu.SemaphoreType.DMA,
        ],
    )
    def k(x_hbm, b_hbm, sums_hbm, cnts_hbm, gbuf, ebuf, idxbuf, acc, cnt,
          areg, smem, sem0, sem1):
        cid = lax.axis_index("c")
        sid = lax.axis_index("s")
        wid = sid * NC + cid
        is_lo = wid < N_LO
        nsg = jnp.where(is_lo, SG_LO, SG_HI)
        base_row = wid * ROWS_LO - 32 * jnp.maximum(wid - N_LO, 0)
        sems = (sem0, sem1)
        is_tail = wid == NW - 1

        pltpu.async_copy(x_hbm.at[pl.ds(base_row, 32)], gbuf.at[0], sem0)

        @pl.when(is_lo)
        def _():
            pltpu.sync_copy(b_hbm.at[pl.ds(base_row, ROWS_LO)],
                            idxbuf.at[pl.ds(0, ROWS_LO)])

        @pl.when(jnp.logical_not(is_lo))
        def _():
            pltpu.sync_copy(b_hbm.at[pl.ds(base_row, SG_HI * 32)],
                            idxbuf.at[pl.ds(0, SG_HI * 32)])

        @pl.when(is_tail)
        def _():
            pltpu.sync_copy(x_hbm.at[pl.ds(TAIL_BASE, TAIL)], ebuf)
            pltpu.sync_copy(b_hbm.at[pl.ds(TAIL_BASE, TAIL)],
                            idxbuf.at[pl.ds(ROWS_LO, TAIL)])

        zv = jnp.zeros((LANES,), jnp.float32)
        ones16 = jnp.ones((LANES,), jnp.float32)

        def zero_body(r, carry):
            for c in range(COLB):
                acc[r, pl.ds(c * LANES, LANES)] = zv
            cnt[r] = zv
            return carry

        lax.fori_loop(0, NUM_GRAPHS, zero_body, 0)
        for c in range(COLB + 1):
            areg[c] = zv

        def flush_to_mem(cur):
            cnt[cur] = cnt[cur] + areg[COLB]
            for c in range(COLB):
                sl = pl.ds(c * LANES, LANES)
                acc[cur, sl] = acc[cur, sl] + areg[c]
            for c in range(COLB + 1):
                areg[c] = zv

        def scatter_rows(segs, row_at):
            for i in range(16):
                s = segs[i]
                cnt[s] = cnt[s] + ones16
                for c in range(COLB):
                    sl = pl.ds(c * LANES, LANES)
                    acc[s, sl] = acc[s, sl] + row_at(i, c)

        def process_sg(sg, b):
            segsA = idxbuf[pl.ds(sg * 32, 16)]
            segsB = idxbuf[pl.ds(sg * 32 + 16, 16)]
            cur = smem[0]
            s0 = segsA[0]
            s31 = segsB[15]
            fast = jnp.logical_and(s0 == cur, s0 == s31)

            @pl.when(fast)
            def _():
                for c in range(COLB):
                    sl = pl.ds(c * LANES, LANES)
                    v = [gbuf[b, i, sl] for i in range(32)]
                    while len(v) > 1:
                        v = [v[2 * j] + v[2 * j + 1]
                             for j in range(len(v) // 2)]
                    areg[c] = areg[c] + v[0]
                areg[COLB] = areg[COLB] + 32.0

            @pl.when(jnp.logical_not(fast))
            def _():
                flush_to_mem(cur)
                scatter_rows(segsA, lambda i, c:
                             gbuf[b, i, pl.ds(c * LANES, LANES)])
                scatter_rows(segsB, lambda i, c:
                             gbuf[b, 16 + i, pl.ds(c * LANES, LANES)])

            smem[0] = s31

        def ring_body(t, carry):
            for b in range(2):
                sg = 2 * t + b

                @pl.when(sg < nsg)
                def _(sg=sg, b=b):
                    @pl.when(sg + 1 < nsg)
                    def _():
                        pltpu.async_copy(
                            x_hbm.at[pl.ds(base_row + (sg + 1) * 32, 32)],
                            gbuf.at[1 - b], sems[1 - b])

                    pltpu.make_async_copy(
                        x_hbm.at[pl.ds(0, 32)], gbuf.at[b], sems[b]).wait()
                    process_sg(sg, b)

            return carry

        segs0 = idxbuf[pl.ds(0, 16)]
        smem[0] = segs0[0]
        lax.fori_loop(0, SG_LO // 2, ring_body, 0)

        @pl.when(is_tail)
        def _():
            segsE = idxbuf[pl.ds(ROWS_LO, TAIL)]
            flush_to_mem(smem[0])
            scatter_rows(segsE, lambda i, c:
                         ebuf[i, pl.ds(c * LANES, LANES)])

        flush_to_mem(smem[0])

        pltpu.sync_copy(acc, sums_hbm.at[wid])
        pltpu.sync_copy(cnt, cnts_hbm.at[wid])

    return k(x, batch_i32)


def _tc_head(psums, pcnts, u, w1, b1, w2, b2):
    def body(ps_ref, pc_ref, u_ref, w1_ref, b1_ref, w2_ref, b2_ref, o_ref):
        sums = jnp.sum(ps_ref[...], axis=0)
        cnts = jnp.sum(pc_ref[...], axis=0)[:, 0:1]
        mean = sums / jnp.maximum(cnts, 1.0)
        w1u = w1_ref[0:GLOBAL_NF, :]
        w1m = w1_ref[GLOBAL_NF:, :]
        h = jnp.dot(u_ref[...], w1u, preferred_element_type=jnp.float32)
        h = h + jnp.dot(mean, w1m, preferred_element_type=jnp.float32)
        h = jnp.maximum(h + b1_ref[...], 0.0)
        o_ref[...] = (jnp.dot(h, w2_ref[...], preferred_element_type=jnp.float32)
                      + b2_ref[...])

    return pl.pallas_call(
        body,
        out_shape=jax.ShapeDtypeStruct((NUM_GRAPHS, GLOBAL_NF), jnp.float32),
    )(psums, pcnts, u, w1, b1, w2, b2)


def kernel(x, edge_index, edge_attr, u, batch, W1, b1, W2, b2):
    batch_i32 = batch.astype(jnp.int32)
    psums, pcnts = _sc_segment_partials(x, batch_i32)
    return _tc_head(psums, pcnts, u, W1,
                    b1.reshape(1, HIDDEN_NF), W2, b2.reshape(1, GLOBAL_NF))


# speedup vs baseline: 1.2428x; 1.2428x over previous
import functools

import jax
import jax.numpy as jnp
from jax import lax
from jax.experimental import pallas as pl
from jax.experimental.pallas import tpu as pltpu
from jax.experimental.pallas import tpu_sc as plsc

NUM_NODES = 10000
NODE_NF = 128
GLOBAL_NF = 64
HIDDEN_NF = 256
NUM_GRAPHS = 64

NC = 2
NS = 16
NW = NC * NS
LANES = 16
COLB = NODE_NF // LANES

N_LO = 17
G_LO = 20
G_HI = 19
ROWS_LO = G_LO * 16
ROWS_HI = G_HI * 16


def _sc_segment_partials(x, batch_i32):
    mesh = plsc.VectorSubcoreMesh(core_axis_name="c", subcore_axis_name="s")

    @functools.partial(
        pl.kernel,
        mesh=mesh,
        out_type=[
            jax.ShapeDtypeStruct((NW, NUM_GRAPHS, NODE_NF), jnp.float32),
            jax.ShapeDtypeStruct((NW, NUM_GRAPHS, LANES), jnp.float32),
        ],
        scratch_types=[
            pltpu.VMEM((ROWS_LO, NODE_NF), jnp.float32),
            pltpu.VMEM((ROWS_LO,), jnp.int32),
            pltpu.VMEM((NUM_GRAPHS, NODE_NF), jnp.float32),
            pltpu.VMEM((NUM_GRAPHS, LANES), jnp.float32),
            pltpu.VMEM((COLB + 1, LANES), jnp.float32),
            pltpu.SMEM((8,), jnp.int32),
            pltpu.SemaphoreType.DMA,
        ],
    )
    def k(x_hbm, b_hbm, sums_hbm, cnts_hbm, xbuf, idxbuf, acc, cnt, areg,
          smem, sem0):
        cid = lax.axis_index("c")
        sid = lax.axis_index("s")
        wid = sid * NC + cid
        is_lo = wid < N_LO
        ngroups = jnp.where(is_lo, G_LO, G_HI)
        base_row = wid * ROWS_LO - 16 * jnp.maximum(wid - N_LO, 0)

        @pl.when(is_lo)
        def _():
            pltpu.async_copy(x_hbm.at[pl.ds(base_row, ROWS_LO)], xbuf, sem0)

        @pl.when(jnp.logical_not(is_lo))
        def _():
            pltpu.async_copy(x_hbm.at[pl.ds(base_row, ROWS_HI)],
                             xbuf.at[pl.ds(0, ROWS_HI)], sem0)

        zv = jnp.zeros((LANES,), jnp.float32)

        def zero_body(r, carry):
            for c in range(COLB):
                acc[r, pl.ds(c * LANES, LANES)] = zv
            cnt[r] = zv
            return carry

        lax.fori_loop(0, NUM_GRAPHS, zero_body, 0)
        for c in range(COLB + 1):
            areg[c] = zv

        @pl.when(is_lo)
        def _():
            pltpu.sync_copy(b_hbm.at[pl.ds(base_row, ROWS_LO)], idxbuf)
            pltpu.make_async_copy(x_hbm.at[pl.ds(0, ROWS_LO)], xbuf,
                                  sem0).wait()

        @pl.when(jnp.logical_not(is_lo))
        def _():
            pltpu.sync_copy(b_hbm.at[pl.ds(base_row, ROWS_HI)],
                            idxbuf.at[pl.ds(0, ROWS_HI)])
            pltpu.make_async_copy(x_hbm.at[pl.ds(0, ROWS_HI)],
                                  xbuf.at[pl.ds(0, ROWS_HI)], sem0).wait()

        def flush_to_mem(cur):
            cnt[cur] = cnt[cur] + areg[COLB]
            for c in range(COLB):
                sl = pl.ds(c * LANES, LANES)
                acc[cur, sl] = acc[cur, sl] + areg[c]
            for c in range(COLB + 1):
                areg[c] = zv

        def group_body(g, carry):
            segs = idxbuf[pl.ds(g * 16, 16)]
            cur = smem[0]
            s0 = segs[0]
            s15 = segs[15]
            fast = jnp.logical_and(s0 == cur, s0 == s15)
            r0 = g * 16

            @pl.when(fast)
            def _():
                for c in range(COLB):
                    sl = pl.ds(c * LANES, LANES)
                    v = [xbuf[r0 + i, sl] for i in range(16)]
                    while len(v) > 1:
                        v = [v[2 * j] + v[2 * j + 1]
                             for j in range(len(v) // 2)]
                    areg[c] = areg[c] + v[0]
                areg[COLB] = areg[COLB] + 16.0

            @pl.when(jnp.logical_not(fast))
            def _():
                cur_ = cur
                for i in range(16):
                    s = segs[i]

                    @pl.when(s != cur_)
                    def _(cur_=cur_):
                        flush_to_mem(cur_)

                    for c in range(COLB):
                        sl = pl.ds(c * LANES, LANES)
                        areg[c] = areg[c] + xbuf[r0 + i, sl]
                    areg[COLB] = areg[COLB] + 1.0
                    cur_ = s

            smem[0] = s15
            return carry

        segs0 = idxbuf[pl.ds(0, 16)]
        smem[0] = segs0[0]
        lax.fori_loop(0, ngroups, group_body, 0)
        flush_to_mem(smem[0])

        pltpu.sync_copy(acc, sums_hbm.at[wid])
        pltpu.sync_copy(cnt, cnts_hbm.at[wid])

    return k(x, batch_i32)


def _tc_head(psums, pcnts, u, w1, b1, w2, b2):
    def body(ps_ref, pc_ref, u_ref, w1_ref, b1_ref, w2_ref, b2_ref, o_ref):
        sums = jnp.sum(ps_ref[...], axis=0)
        cnts = jnp.sum(pc_ref[...], axis=0)[:, 0:1]
        mean = sums / jnp.maximum(cnts, 1.0)
        w1u = w1_ref[0:GLOBAL_NF, :]
        w1m = w1_ref[GLOBAL_NF:, :]
        h = jnp.dot(u_ref[...], w1u, preferred_element_type=jnp.float32)
        h = h + jnp.dot(mean, w1m, preferred_element_type=jnp.float32)
        h = jnp.maximum(h + b1_ref[...], 0.0)
        o_ref[...] = (jnp.dot(h, w2_ref[...], preferred_element_type=jnp.float32)
                      + b2_ref[...])

    return pl.pallas_call(
        body,
        out_shape=jax.ShapeDtypeStruct((NUM_GRAPHS, GLOBAL_NF), jnp.float32),
    )(psums, pcnts, u, w1, b1, w2, b2)


def kernel(x, edge_index, edge_attr, u, batch, W1, b1, W2, b2):
    batch_i32 = batch.astype(jnp.int32)
    psums, pcnts = _sc_segment_partials(x, batch_i32)
    return _tc_head(psums, pcnts, u, W1,
                    b1.reshape(1, HIDDEN_NF), W2, b2.reshape(1, GLOBAL_NF))
